# baseline (speedup 1.0000x reference)
import functools

import jax
import jax.numpy as jnp
from jax import lax
from jax.experimental import pallas as pl
from jax.experimental.pallas import tpu as pltpu
from jax.experimental.pallas import tpu_sc as plsc

H, W = 48, 96
M = H * W
N = 4096
BS = 2
C = 256
P = BS * N
Q = BS * M
NW = 32

MT = 512
NCH = 1024
NSTEPS = M // MT

K2_PER_W = P // NW
K2_CH = 32
K2_NCH = K2_PER_W // K2_CH
K2_NCHT = P // K2_CH

K3_PER_W = Q // NW
K3_CH = 48
K3_NCH = K3_PER_W // K3_CH

DMT = 512
DNT = Q // DMT


def _meta_body(uv_ref, w4_ref, lin4_ref):
    x = uv_ref[0, 0:1, :]
    y = uv_ref[0, 1:2, :]
    x0 = jnp.floor(x)
    y0 = jnp.floor(y)
    for k, (dy, dx) in enumerate(((0, 0), (0, 1), (1, 0), (1, 1))):
        xi = x0 + float(dx)
        yi = y0 + float(dy)
        wgt = (1.0 - jnp.abs(x - xi)) * (1.0 - jnp.abs(y - yi))
        valid = (xi >= 0) & (xi <= W - 1) & (yi >= 0) & (yi <= H - 1)
        wgt = wgt * valid.astype(jnp.float32)
        xi_c = jnp.clip(xi, 0, W - 1).astype(jnp.int32)
        yi_c = jnp.clip(yi, 0, H - 1).astype(jnp.int32)
        lin = yi_c * W + xi_c
        w4_ref[0, k : k + 1, :] = wgt
        lin4_ref[0, k : k + 1, :] = lin


def _meta_stage(uv):
    return pl.pallas_call(
        _meta_body,
        grid=(BS,),
        in_specs=[pl.BlockSpec((1, 2, N), lambda b: (b, 0, 0))],
        out_specs=[
            pl.BlockSpec((1, 4, N), lambda b: (b, 0, 0)),
            pl.BlockSpec((1, 4, N), lambda b: (b, 0, 0)),
        ],
        out_shape=[
            jax.ShapeDtypeStruct((BS, 4, N), jnp.float32),
            jax.ShapeDtypeStruct((BS, 4, N), jnp.int32),
        ],
    )(uv)


def _knn_body(uvt_ref, nn_ref):
    t = pl.program_id(1)

    lane = lax.broadcasted_iota(jnp.int32, (1, MT), 1) + t * MT
    xm = (lane % W).astype(jnp.float32)
    ym = (lane // W).astype(jnp.float32)
    q2 = xm * xm + ym * ym
    gxy2 = jnp.concatenate([2.0 * xm, 2.0 * ym], axis=0).astype(jnp.bfloat16)

    best = jnp.full((1, MT), jnp.inf, dtype=jnp.float32)
    bidx = jnp.zeros((1, MT), dtype=jnp.int32)
    for c in range(N // NCH):
        uvc = uvt_ref[0, pl.ds(c * NCH, NCH), :]
        u = uvc[:, 0:1]
        v = uvc[:, 1:2]
        i2 = u * u + v * v
        s1 = uvc.astype(jnp.bfloat16)
        t1 = jnp.dot(s1, gxy2, preferred_element_type=jnp.float32)
        d = (q2 + i2) - t1
        cmin = jnp.min(d, axis=0, keepdims=True)
        riota = lax.broadcasted_iota(jnp.int32, (NCH, MT), 0) + c * NCH
        cidx = jnp.min(jnp.where(d == cmin, riota, N), axis=0, keepdims=True)
        take = cmin < best
        best = jnp.where(take, cmin, best)
        bidx = jnp.where(take, cidx, bidx)
    nn_ref[0] = bidx


def _knn_stage(uvt):
    nn3 = pl.pallas_call(
        _knn_body,
        grid=(BS, NSTEPS),
        in_specs=[pl.BlockSpec((1, N, 2), lambda b, t: (b, 0, 0))],
        out_specs=pl.BlockSpec((1, 1, MT), lambda b, t: (b * NSTEPS + t, 0, 0)),
        out_shape=jax.ShapeDtypeStruct((BS * NSTEPS, 1, MT), jnp.int32),
    )(uvt)
    return nn3.reshape(BS, M)


_mesh = functools.partial(plsc.VectorSubcoreMesh,
                          core_axis_name="c", subcore_axis_name="s")


def _wid():
    return lax.axis_index("s") * 2 + lax.axis_index("c")


def _k2_body(f2t_hbm, lingp_hbm, w4p_hbm, s_hbm,
             idx0, idx1, g00, g01, g02, g03, g10, g11, g12, g13,
             wb0, wb1, acc0, acc1, sem0, sem1):
    wid = _wid()
    chunk0 = wid * K2_NCH

    set0 = (idx0, (g00, g01, g02, g03), wb0, acc0, sem0)
    set1 = (idx1, (g10, g11, g12, g13), wb1, acc1, sem1)

    def issue(ci, s):
        idxall, gs, wbuf, _, sem = s
        cid = chunk0 + ci
        pltpu.sync_copy(lingp_hbm.at[cid], idxall)
        pltpu.sync_copy(w4p_hbm.at[cid], wbuf)
        for k in range(4):
            pltpu.async_copy(
                f2t_hbm.at[idxall.at[pl.ds(k * K2_CH, K2_CH)]], gs[k], sem)

    def drain(s):
        _, gs, _, _, sem = s
        for k in range(4):
            pltpu.make_async_copy(f2t_hbm.at[pl.ds(0, K2_CH)], gs[k], sem).wait()

    def compute(ci, s):
        _, gs, wbuf, acc, _ = s
        g0, g1, g2, g3 = gs

        @plsc.parallel_loop(0, K2_CH, unroll=8)
        def row(r):
            wv0 = wbuf[0, r, :]
            wv1 = wbuf[1, r, :]
            wv2 = wbuf[2, r, :]
            wv3 = wbuf[3, r, :]
            for j in range(C // 16):
                s_ = (g0[r, pl.ds(j * 16, 16)] * wv0
                      + g1[r, pl.ds(j * 16, 16)] * wv1
                      + g2[r, pl.ds(j * 16, 16)] * wv2
                      + g3[r, pl.ds(j * 16, 16)] * wv3)
                acc[r, pl.ds(j * 16, 16)] = s_
        pltpu.sync_copy(acc, s_hbm.at[pl.ds((chunk0 + ci) * K2_CH, K2_CH)])

    issue(0, set0)

    def pair(cp, carry):
        ci0 = cp * 2
        drain(set0)
        issue(ci0 + 1, set1)
        compute(ci0, set0)
        drain(set1)

        @pl.when(ci0 + 2 < K2_NCH)
        def _():
            issue(ci0 + 2, set0)

        compute(ci0 + 1, set1)
        return carry

    lax.fori_loop(0, K2_NCH // 2, pair, 0)


def _sampled_stage(f2t, lingp, w4p):
    gbuf = lambda: pltpu.VMEM((K2_CH, C), jnp.float32)
    kfn = pl.kernel(
        _k2_body,
        mesh=_mesh(),
        out_type=jax.ShapeDtypeStruct((P, C), jnp.float32),
        scratch_types=[
            pltpu.VMEM((4 * K2_CH,), jnp.int32),
            pltpu.VMEM((4 * K2_CH,), jnp.int32),
            gbuf(), gbuf(), gbuf(), gbuf(),
            gbuf(), gbuf(), gbuf(), gbuf(),
            pltpu.VMEM((4, K2_CH, 16), jnp.float32),
            pltpu.VMEM((4, K2_CH, 16), jnp.float32),
            gbuf(), gbuf(),
            pltpu.SemaphoreType.DMA,
            pltpu.SemaphoreType.DMA,
        ],
    )
    return kfn(f2t, lingp, w4p)


def _k3_body(f3uv_hbm, s_hbm, nng_hbm, gf3uv_hbm, gs_hbm,
             idx0, idx1, b10, b20, b11, b21, sem0, sem1):
    wid = _wid()
    base0 = wid * K3_PER_W

    sets = ((idx0, b10, b20, sem0), (idx1, b11, b21, sem1))

    def issue(ci, s):
        idxv, b1, b2, sem = s
        base = base0 + ci * K3_CH
        pltpu.sync_copy(nng_hbm.at[pl.ds(base, K3_CH)], idxv)
        return [pltpu.async_copy(f3uv_hbm.at[idxv], b1, sem),
                pltpu.async_copy(s_hbm.at[idxv], b2, sem)]

    handles = {0: issue(0, sets[0])}
    for ci in range(K3_NCH):
        cur = ci % 2
        _, b1, b2, _ = sets[cur]
        for h in handles.pop(ci):
            h.wait()
        if ci + 1 < K3_NCH:
            handles[ci + 1] = issue(ci + 1, sets[1 - cur])
        base = base0 + ci * K3_CH
        pltpu.sync_copy(b1, gf3uv_hbm.at[pl.ds(base, K3_CH)])
        pltpu.sync_copy(b2, gs_hbm.at[pl.ds(base, K3_CH)])


def _nn_gather_stage(f3uv, s, nng):
    kfn = pl.kernel(
        _k3_body,
        mesh=_mesh(),
        out_type=[
            jax.ShapeDtypeStruct((Q, 384), jnp.float32),
            jax.ShapeDtypeStruct((Q, C), jnp.float32),
        ],
        scratch_types=[
            pltpu.VMEM((K3_CH,), jnp.int32),
            pltpu.VMEM((K3_CH,), jnp.int32),
            pltpu.VMEM((K3_CH, 384), jnp.float32),
            pltpu.VMEM((K3_CH, C), jnp.float32),
            pltpu.VMEM((K3_CH, 384), jnp.float32),
            pltpu.VMEM((K3_CH, C), jnp.float32),
            pltpu.SemaphoreType.DMA,
            pltpu.SemaphoreType.DMA,
        ],
    )
    return kfn(f3uv, s, nng)


def _dense_body(g1_ref, gs_ref, f2_ref, w1t3_ref, w1h_ref, b1_ref,
                w2t_ref, b2_ref, w3t_ref, b3_ref, out_ref):
    t = pl.program_id(0)
    m0 = (t % (M // DMT)) * DMT

    g1 = g1_ref[...]
    gs = gs_ref[...]
    f2 = f2_ref[...]

    corr = jnp.sum(gs * f2, axis=1, keepdims=True) * (1.0 / C)

    mi = lax.broadcasted_iota(jnp.int32, (DMT, 1), 0) + m0
    gx = (mi % W).astype(jnp.float32)
    gy = (mi // W).astype(jnp.float32)
    off_x = g1[:, 256:257] - gx
    off_y = g1[:, 257:258] - gy

    f3 = g1[:, 0:256]
    x = jnp.dot(f3, w1t3_ref[...], preferred_element_type=jnp.float32)
    x = x + off_x * w1h_ref[0:1, :] + off_y * w1h_ref[1:2, :] \
          + corr * w1h_ref[2:3, :] + b1_ref[...]
    x = jnp.where(x >= 0, x, 0.1 * x)
    x = jnp.dot(x, w2t_ref[...], preferred_element_type=jnp.float32) + b2_ref[...]
    x = jnp.where(x >= 0, x, 0.1 * x)
    x = jnp.dot(x, w3t_ref[...], preferred_element_type=jnp.float32) + b3_ref[...]
    x = jnp.where(x >= 0, x, 0.1 * x)
    out_ref[...] = x


def _dense_stage(g1, gs, f2t, W1, b1, W2, b2, W3, b3):
    w1t3 = jnp.transpose(W1[:, 3:])
    w1h = W1[:, 0:3].T
    w2t = jnp.transpose(W2)
    w3t = jnp.transpose(W3)
    full = lambda shape: pl.BlockSpec(shape, lambda t: tuple(0 for _ in shape))
    return pl.pallas_call(
        _dense_body,
        grid=(DNT,),
        in_specs=[
            pl.BlockSpec((DMT, 384), lambda t: (t, 0)),
            pl.BlockSpec((DMT, 256), lambda t: (t, 0)),
            pl.BlockSpec((DMT, 256), lambda t: (t, 0)),
            full((C, C)), full((3, C)), full((1, C)),
            full((C, C)), full((1, C)),
            full((C, C)), full((1, C)),
        ],
        out_specs=pl.BlockSpec((DMT, 256), lambda t: (t, 0)),
        out_shape=jax.ShapeDtypeStruct((Q, 256), jnp.float32),
    )(g1, gs, f2t, w1t3, w1h, b1.reshape(1, C), w2t, b2.reshape(1, C),
      w3t, b3.reshape(1, C))


def kernel(uv, feat_2d, feat_3d, W1, b1, W2, b2, W3, b3):
    bs, c, h, w = feat_2d.shape

    uvt = jnp.transpose(uv, (0, 2, 1))
    f2t = feat_2d.reshape(bs, c, M).transpose(0, 2, 1).reshape(Q, C)
    f3uv = jnp.concatenate([
        feat_3d.transpose(0, 2, 1),
        uvt,
        jnp.zeros((bs, N, 126), jnp.float32),
    ], axis=2).reshape(P, 384)

    w4, lin4 = _meta_stage(uv)

    boff_m = (jnp.arange(BS, dtype=jnp.int32) * M)[:, None, None]
    ling = (lin4 + boff_m).transpose(1, 0, 2).reshape(4, P)
    lingp = ling.reshape(4, K2_NCHT, K2_CH).transpose(1, 0, 2).reshape(
        K2_NCHT, 4 * K2_CH)
    w4p = jnp.broadcast_to(
        w4.transpose(1, 0, 2).reshape(4, P)[:, :, None], (4, P, 16))
    w4p = w4p.reshape(4, K2_NCHT, K2_CH, 16).transpose(1, 0, 2, 3)

    s = _sampled_stage(f2t, lingp, w4p)

    nn_idx = _knn_stage(uvt)
    nng = (nn_idx + (jnp.arange(BS, dtype=jnp.int32) * N)[:, None]).reshape(Q)
    g1, gs = _nn_gather_stage(f3uv, s, nng)

    out = _dense_stage(g1, gs, f2t, W1, b1, W2, b2, W3, b3)
    return out.reshape(bs, M, C).transpose(0, 2, 1).reshape(bs, C, h, w)


# speedup vs baseline: 1.0246x; 1.0246x over previous
import functools

import jax
import jax.numpy as jnp
from jax import lax
from jax.experimental import pallas as pl
from jax.experimental.pallas import tpu as pltpu
from jax.experimental.pallas import tpu_sc as plsc

H, W = 48, 96
M = H * W
N = 4096
BS = 2
C = 256
P = BS * N
Q = BS * M
NW = 32

MT = 512
NCH = 512
NSTEPS = M // MT

K2_PER_W = P // NW
K2_CH = 32
K2_NCH = K2_PER_W // K2_CH
K2_NCHT = P // K2_CH

K3_PER_W = Q // NW
K3_CH = 48
K3_NCH = K3_PER_W // K3_CH

DMT = 512
DNT = Q // DMT


def _meta_body(uv_ref, w4_ref, lin4_ref):
    x = uv_ref[0, 0:1, :]
    y = uv_ref[0, 1:2, :]
    x0 = jnp.floor(x)
    y0 = jnp.floor(y)
    for k, (dy, dx) in enumerate(((0, 0), (0, 1), (1, 0), (1, 1))):
        xi = x0 + float(dx)
        yi = y0 + float(dy)
        wgt = (1.0 - jnp.abs(x - xi)) * (1.0 - jnp.abs(y - yi))
        valid = (xi >= 0) & (xi <= W - 1) & (yi >= 0) & (yi <= H - 1)
        wgt = wgt * valid.astype(jnp.float32)
        xi_c = jnp.clip(xi, 0, W - 1).astype(jnp.int32)
        yi_c = jnp.clip(yi, 0, H - 1).astype(jnp.int32)
        lin = yi_c * W + xi_c
        w4_ref[0, k : k + 1, :] = wgt
        lin4_ref[0, k : k + 1, :] = lin


def _meta_stage(uv):
    return pl.pallas_call(
        _meta_body,
        grid=(BS,),
        in_specs=[pl.BlockSpec((1, 2, N), lambda b: (b, 0, 0))],
        out_specs=[
            pl.BlockSpec((1, 4, N), lambda b: (b, 0, 0)),
            pl.BlockSpec((1, 4, N), lambda b: (b, 0, 0)),
        ],
        out_shape=[
            jax.ShapeDtypeStruct((BS, 4, N), jnp.float32),
            jax.ShapeDtypeStruct((BS, 4, N), jnp.int32),
        ],
    )(uv)


def _knn_body(uvt_ref, nn_ref):
    t = pl.program_id(1)

    lane = lax.broadcasted_iota(jnp.int32, (1, MT), 1) + t * MT
    xm = (lane % W).astype(jnp.float32)
    ym = (lane // W).astype(jnp.float32)
    q2 = xm * xm + ym * ym
    gxy2 = jnp.concatenate([2.0 * xm, 2.0 * ym], axis=0).astype(jnp.bfloat16)

    best = jnp.full((1, MT), jnp.inf, dtype=jnp.float32)
    bidx = jnp.zeros((1, MT), dtype=jnp.int32)
    for c in range(N // NCH):
        uvc = uvt_ref[0, pl.ds(c * NCH, NCH), :]
        u = uvc[:, 0:1]
        v = uvc[:, 1:2]
        i2 = u * u + v * v
        s1 = uvc.astype(jnp.bfloat16)
        t1 = jnp.dot(s1, gxy2, preferred_element_type=jnp.float32)
        d = (q2 + i2) - t1
        cmin = jnp.min(d, axis=0, keepdims=True)
        riota = lax.broadcasted_iota(jnp.int32, (NCH, MT), 0) + c * NCH
        cidx = jnp.min(jnp.where(d == cmin, riota, N), axis=0, keepdims=True)
        take = cmin < best
        best = jnp.where(take, cmin, best)
        bidx = jnp.where(take, cidx, bidx)
    nn_ref[0] = bidx


def _knn_stage(uvt):
    nn3 = pl.pallas_call(
        _knn_body,
        grid=(BS, NSTEPS),
        in_specs=[pl.BlockSpec((1, N, 2), lambda b, t: (b, 0, 0))],
        out_specs=pl.BlockSpec((1, 1, MT), lambda b, t: (b * NSTEPS + t, 0, 0)),
        out_shape=jax.ShapeDtypeStruct((BS * NSTEPS, 1, MT), jnp.int32),
    )(uvt)
    return nn3.reshape(BS, M)


_mesh = functools.partial(plsc.VectorSubcoreMesh,
                          core_axis_name="c", subcore_axis_name="s")


def _wid():
    return lax.axis_index("s") * 2 + lax.axis_index("c")


def _k2_body(f2t_hbm, lingp_hbm, w4p_hbm, s_hbm,
             idx0, idx1, g00, g01, g02, g03, g10, g11, g12, g13,
             wb0, wb1, acc0, acc1, sem0, sem1):
    wid = _wid()
    chunk0 = wid * K2_NCH

    set0 = (idx0, (g00, g01, g02, g03), wb0, acc0, sem0)
    set1 = (idx1, (g10, g11, g12, g13), wb1, acc1, sem1)

    def issue(ci, s):
        idxall, gs, wbuf, _, sem = s
        cid = chunk0 + ci
        pltpu.sync_copy(lingp_hbm.at[cid], idxall)
        pltpu.sync_copy(w4p_hbm.at[cid], wbuf)
        for k in range(4):
            pltpu.async_copy(
                f2t_hbm.at[idxall.at[pl.ds(k * K2_CH, K2_CH)]], gs[k], sem)

    def drain(s):
        _, gs, _, _, sem = s
        for k in range(4):
            pltpu.make_async_copy(f2t_hbm.at[pl.ds(0, K2_CH)], gs[k], sem).wait()

    def compute(ci, s):
        _, gs, wbuf, acc, _ = s
        g0, g1, g2, g3 = gs

        @plsc.parallel_loop(0, K2_CH, unroll=4)
        def row(r):
            wv0 = wbuf[0, r, :]
            wv1 = wbuf[1, r, :]
            wv2 = wbuf[2, r, :]
            wv3 = wbuf[3, r, :]
            for j in range(C // 16):
                s_ = (g0[r, pl.ds(j * 16, 16)] * wv0
                      + g1[r, pl.ds(j * 16, 16)] * wv1
                      + g2[r, pl.ds(j * 16, 16)] * wv2
                      + g3[r, pl.ds(j * 16, 16)] * wv3)
                acc[r, pl.ds(j * 16, 16)] = s_
        pltpu.sync_copy(acc, s_hbm.at[pl.ds((chunk0 + ci) * K2_CH, K2_CH)])

    issue(0, set0)

    def pair(cp, carry):
        ci0 = cp * 2
        drain(set0)
        issue(ci0 + 1, set1)
        compute(ci0, set0)
        drain(set1)

        @pl.when(ci0 + 2 < K2_NCH)
        def _():
            issue(ci0 + 2, set0)

        compute(ci0 + 1, set1)
        return carry

    lax.fori_loop(0, K2_NCH // 2, pair, 0)


def _sampled_stage(f2t, lingp, w4p):
    gbuf = lambda: pltpu.VMEM((K2_CH, C), jnp.float32)
    kfn = pl.kernel(
        _k2_body,
        mesh=_mesh(),
        out_type=jax.ShapeDtypeStruct((P, C), jnp.float32),
        scratch_types=[
            pltpu.VMEM((4 * K2_CH,), jnp.int32),
            pltpu.VMEM((4 * K2_CH,), jnp.int32),
            gbuf(), gbuf(), gbuf(), gbuf(),
            gbuf(), gbuf(), gbuf(), gbuf(),
            pltpu.VMEM((4, K2_CH, 16), jnp.float32),
            pltpu.VMEM((4, K2_CH, 16), jnp.float32),
            gbuf(), gbuf(),
            pltpu.SemaphoreType.DMA,
            pltpu.SemaphoreType.DMA,
        ],
    )
    return kfn(f2t, lingp, w4p)


def _k3_body(f3uv_hbm, s_hbm, nng_hbm, gf3uv_hbm, gs_hbm,
             idx0, idx1, b10, b20, b11, b21, sem0, sem1):
    wid = _wid()
    base0 = wid * K3_PER_W

    sets = ((idx0, b10, b20, sem0), (idx1, b11, b21, sem1))

    def issue(ci, s):
        idxv, b1, b2, sem = s
        base = base0 + ci * K3_CH
        pltpu.sync_copy(nng_hbm.at[pl.ds(base, K3_CH)], idxv)
        return [pltpu.async_copy(f3uv_hbm.at[idxv], b1, sem),
                pltpu.async_copy(s_hbm.at[idxv], b2, sem)]

    handles = {0: issue(0, sets[0])}
    for ci in range(K3_NCH):
        cur = ci % 2
        _, b1, b2, _ = sets[cur]
        for h in handles.pop(ci):
            h.wait()
        if ci + 1 < K3_NCH:
            handles[ci + 1] = issue(ci + 1, sets[1 - cur])
        base = base0 + ci * K3_CH
        pltpu.sync_copy(b1, gf3uv_hbm.at[pl.ds(base, K3_CH)])
        pltpu.sync_copy(b2, gs_hbm.at[pl.ds(base, K3_CH)])


def _nn_gather_stage(f3uv, s, nng):
    kfn = pl.kernel(
        _k3_body,
        mesh=_mesh(),
        out_type=[
            jax.ShapeDtypeStruct((Q, 384), jnp.float32),
            jax.ShapeDtypeStruct((Q, C), jnp.float32),
        ],
        scratch_types=[
            pltpu.VMEM((K3_CH,), jnp.int32),
            pltpu.VMEM((K3_CH,), jnp.int32),
            pltpu.VMEM((K3_CH, 384), jnp.float32),
            pltpu.VMEM((K3_CH, C), jnp.float32),
            pltpu.VMEM((K3_CH, 384), jnp.float32),
            pltpu.VMEM((K3_CH, C), jnp.float32),
            pltpu.SemaphoreType.DMA,
            pltpu.SemaphoreType.DMA,
        ],
    )
    return kfn(f3uv, s, nng)


def _dense_body(g1_ref, gs_ref, f2_ref, w1t3_ref, w1h_ref, b1_ref,
                w2t_ref, b2_ref, w3t_ref, b3_ref, out_ref):
    t = pl.program_id(0)
    m0 = (t % (M // DMT)) * DMT

    g1 = g1_ref[...]
    gs = gs_ref[...]
    f2 = f2_ref[...]

    corr = jnp.sum(gs * f2, axis=1, keepdims=True) * (1.0 / C)

    mi = lax.broadcasted_iota(jnp.int32, (DMT, 1), 0) + m0
    gx = (mi % W).astype(jnp.float32)
    gy = (mi // W).astype(jnp.float32)
    off_x = g1[:, 256:257] - gx
    off_y = g1[:, 257:258] - gy

    f3 = g1[:, 0:256]
    x = jnp.dot(f3, w1t3_ref[...], preferred_element_type=jnp.float32)
    x = x + off_x * w1h_ref[0:1, :] + off_y * w1h_ref[1:2, :] \
          + corr * w1h_ref[2:3, :] + b1_ref[...]
    x = jnp.where(x >= 0, x, 0.1 * x)
    x = jnp.dot(x, w2t_ref[...], preferred_element_type=jnp.float32) + b2_ref[...]
    x = jnp.where(x >= 0, x, 0.1 * x)
    x = jnp.dot(x, w3t_ref[...], preferred_element_type=jnp.float32) + b3_ref[...]
    x = jnp.where(x >= 0, x, 0.1 * x)
    out_ref[...] = x


def _dense_stage(g1, gs, f2t, W1, b1, W2, b2, W3, b3):
    w1t3 = jnp.transpose(W1[:, 3:])
    w1h = W1[:, 0:3].T
    w2t = jnp.transpose(W2)
    w3t = jnp.transpose(W3)
    full = lambda shape: pl.BlockSpec(shape, lambda t: tuple(0 for _ in shape))
    return pl.pallas_call(
        _dense_body,
        grid=(DNT,),
        in_specs=[
            pl.BlockSpec((DMT, 384), lambda t: (t, 0)),
            pl.BlockSpec((DMT, 256), lambda t: (t, 0)),
            pl.BlockSpec((DMT, 256), lambda t: (t, 0)),
            full((C, C)), full((3, C)), full((1, C)),
            full((C, C)), full((1, C)),
            full((C, C)), full((1, C)),
        ],
        out_specs=pl.BlockSpec((DMT, 256), lambda t: (t, 0)),
        out_shape=jax.ShapeDtypeStruct((Q, 256), jnp.float32),
    )(g1, gs, f2t, w1t3, w1h, b1.reshape(1, C), w2t, b2.reshape(1, C),
      w3t, b3.reshape(1, C))


def kernel(uv, feat_2d, feat_3d, W1, b1, W2, b2, W3, b3):
    bs, c, h, w = feat_2d.shape

    uvt = jnp.transpose(uv, (0, 2, 1))
    f2t = feat_2d.reshape(bs, c, M).transpose(0, 2, 1).reshape(Q, C)
    f3uv = jnp.concatenate([
        feat_3d.transpose(0, 2, 1),
        uvt,
        jnp.zeros((bs, N, 126), jnp.float32),
    ], axis=2).reshape(P, 384)

    w4, lin4 = _meta_stage(uv)

    boff_m = (jnp.arange(BS, dtype=jnp.int32) * M)[:, None, None]
    ling = (lin4 + boff_m).transpose(1, 0, 2).reshape(4, P)
    lingp = ling.reshape(4, K2_NCHT, K2_CH).transpose(1, 0, 2).reshape(
        K2_NCHT, 4 * K2_CH)
    w4p = jnp.broadcast_to(
        w4.transpose(1, 0, 2).reshape(4, P)[:, :, None], (4, P, 16))
    w4p = w4p.reshape(4, K2_NCHT, K2_CH, 16).transpose(1, 0, 2, 3)

    s = _sampled_stage(f2t, lingp, w4p)

    nn_idx = _knn_stage(uvt)
    nng = (nn_idx + (jnp.arange(BS, dtype=jnp.int32) * N)[:, None]).reshape(Q)
    g1, gs = _nn_gather_stage(f3uv, s, nng)

    out = _dense_stage(g1, gs, f2t, W1, b1, W2, b2, W3, b3)
    return out.reshape(bs, M, C).transpose(0, 2, 1).reshape(bs, C, h, w)
